# Initial kernel scaffold; baseline (speedup 1.0000x reference)
import jax
import jax.numpy as jnp
from jax.experimental import pallas as pl


def kernel(x, edge_index, edge_type, lin_w, lin_b, fwd_table, bwd_table):
    raise NotImplementedError("write your pallas kernel here")


# speedup vs baseline: 21.8090x; 21.8090x over previous
import functools

import jax
import jax.numpy as jnp
from jax import lax
from jax.experimental import pallas as pl
from jax.experimental.pallas import tpu as pltpu
from jax.experimental.pallas import tpu_sc as plsc

N = 10000
E = 160000
C = 16
R = 64
NC = 2
NS = 16
L = 16
N2 = 10240
CH = 128
NCH = 79
EP = NCH * CH
E2 = NS * EP
NPT = N2 // NS
TRASH = N2 - 1

_MESH = plsc.VectorSubcoreMesh(
    core_axis_name="c", subcore_axis_name="s", num_cores=NC, num_subcores=NS
)
_SC_PARAMS = pltpu.CompilerParams(
    needs_layout_passes=False, use_tc_tiling_on_sc=False
)


@functools.partial(
    pl.kernel,
    out_type=jax.ShapeDtypeStruct((NC, NS, N2 // 128, 128), jnp.float32),
    mesh=_MESH,
    compiler_params=_SC_PARAMS,
    scratch_types=[
        pltpu.VMEM((NCH, CH), jnp.int32),
        pltpu.VMEM((N2 // 128, 128), jnp.float32),
    ],
)
def _sc_counts(ei_hbm, out_hbm, ev, cnt):
    c = lax.axis_index("c")
    s = lax.axis_index("s")
    pltpu.sync_copy(ei_hbm.at[c, s], ev)

    zeros = jnp.zeros((L,), jnp.float32)

    def _zero(i, carry):
        r = i // (128 // L)
        p = i % (128 // L)
        cnt[r, pl.ds(p * L, L)] = zeros
        return carry

    lax.fori_loop(0, N2 // L, _zero, 0)

    ones = jnp.ones((L,), jnp.float32)

    def _accum(i, carry):
        r = i // (CH // L)
        p = i % (CH // L)
        idx = ev[r, pl.ds(p * L, L)]
        plsc.addupdate_scatter(cnt, [idx >> 7, idx & 127], ones)
        return carry

    lax.fori_loop(0, EP // L, _accum, 0)
    pltpu.sync_copy(cnt, out_hbm.at[c, s])


def _norm_body(wf_ref, wb_ref, of_ref, ob_ref):
    for w_ref, o_ref in ((wf_ref, of_ref), (wb_ref, ob_ref)):
        w = w_ref[...]
        nrm = jnp.sqrt(jnp.sum(w * w, axis=1, keepdims=True)) + 0.01
        o_ref[...] = w / nrm


def _tc_norm(fwd_table, bwd_table):
    return pl.pallas_call(
        _norm_body,
        out_shape=(
            jax.ShapeDtypeStruct((R, C * C), jnp.float32),
            jax.ShapeDtypeStruct((R, C * C), jnp.float32),
        ),
    )(fwd_table, bwd_table)


BN = 1024


def _mm_body(x_ref, cp_ref, w2_ref, y_ref):
    x = x_ref[...]
    deg = jnp.sum(cp_ref[...], axis=1) + 1.0
    w2 = w2_ref[...]
    xf = x / deg[0][:, None]
    xb = x / deg[1][:, None]
    y_ref[0] = jnp.dot(xf, w2[0], preferred_element_type=jnp.float32)
    y_ref[1] = jnp.dot(xb, w2[1], preferred_element_type=jnp.float32)


def _tc_matmul(x, cntp, w2):
    return pl.pallas_call(
        _mm_body,
        grid=(N2 // BN,),
        in_specs=[
            pl.BlockSpec((BN, C), lambda i: (i, 0)),
            pl.BlockSpec((NC, NS, BN), lambda i: (0, 0, i)),
            pl.BlockSpec((NC, C, R * C), lambda i: (0, 0, 0)),
        ],
        out_specs=pl.BlockSpec((NC, BN, R * C), lambda i: (0, i, 0)),
        out_shape=jax.ShapeDtypeStruct((NC, N2, R * C), jnp.float32),
    )(x, cntp, w2)


@functools.partial(
    pl.kernel,
    out_type=jax.ShapeDtypeStruct((NC, N2, C), jnp.float32),
    mesh=_MESH,
    compiler_params=_SC_PARAMS,
    scratch_types=[
        pltpu.VMEM((NCH, CH), jnp.int32),
        pltpu.VMEM((NCH, CH), jnp.int32),
        pltpu.VMEM((NCH, CH), jnp.int32),
        pltpu.VMEM((CH, C), jnp.float32),
        pltpu.VMEM((CH, C), jnp.float32),
        pltpu.SemaphoreType.DMA,
        pltpu.VMEM_SHARED((N2, C), jnp.float32),
    ],
)
def _sc_gather_scatter(ei_hbm, et_hbm, y_hbm, out_hbm,
                       gidx, didx, tt, rows, zb, sem, acc):
    c = lax.axis_index("c")
    s = lax.axis_index("s")

    zeros = jnp.zeros((L,), jnp.float32)

    def _zrow(r, carry):
        zb[r, :] = zeros
        return carry

    lax.fori_loop(0, CH, _zrow, 0)
    for k in range(NPT // CH):
        pltpu.sync_copy(zb, acc.at[pl.ds(s * NPT + k * CH, CH)])
    plsc.subcore_barrier()

    pltpu.sync_copy(ei_hbm.at[c, s], gidx)
    pltpu.sync_copy(ei_hbm.at[1 - c, s], didx)
    pltpu.sync_copy(et_hbm.at[s], tt)

    base = c * (N2 * R)

    def _build(r, carry):
        for p in range(CH // L):
            sl = pl.ds(p * L, L)
            gidx[r, sl] = gidx[r, sl] * R + tt[r, sl] + base
        return carry

    lax.fori_loop(0, NCH, _build, 0)

    def _edge_chunk(j, carry):
        pltpu.async_copy(y_hbm.at[gidx.at[j]], rows, sem).wait()
        pltpu.sync_copy(rows, acc.at[didx.at[j]], add=True)
        return carry

    lax.fori_loop(0, NCH, _edge_chunk, 0)

    plsc.subcore_barrier()
    pltpu.sync_copy(acc.at[pl.ds(s * NPT, NPT)],
                    out_hbm.at[c, pl.ds(s * NPT, NPT)])


def _fin_body(x_ref, w_ref, b_ref, acc_ref, o_ref):
    x = x_ref[...]
    lin = lax.dot_general(x, w_ref[...], (((1,), (1,)), ((), ())),
                          preferred_element_type=jnp.float32)
    o_ref[...] = lin + b_ref[...] + acc_ref[0] + acc_ref[1]


def _tc_final(x, lin_w, lin_b2, acc):
    return pl.pallas_call(
        _fin_body,
        out_shape=jax.ShapeDtypeStruct((N, C), jnp.float32),
    )(x, lin_w, lin_b2, acc)


def kernel(x, edge_index, edge_type, lin_w, lin_b, fwd_table, bwd_table):
    pad = jnp.full((2, E2 - E), TRASH, jnp.int32)
    ei3 = jnp.concatenate([edge_index, pad], axis=1).reshape(NC, NS, NCH, CH)
    et3 = jnp.concatenate(
        [edge_type, jnp.zeros((E2 - E,), jnp.int32)]).reshape(NS, NCH, CH)

    cntp = _sc_counts(ei3)

    wnf, wnb = _tc_norm(fwd_table, bwd_table)
    w2 = jnp.stack([
        wnf.reshape(R, C, C).transpose(1, 0, 2).reshape(C, R * C),
        wnb.reshape(R, C, C).transpose(1, 0, 2).reshape(C, R * C),
    ])

    x2 = jnp.concatenate([x, jnp.zeros((N2 - N, C), x.dtype)])
    y = _tc_matmul(x2, cntp.reshape(NC, NS, N2), w2)
    y2 = y.reshape(NC * N2 * R, C)

    acc = _sc_gather_scatter(ei3, et3, y2)

    return _tc_final(x, lin_w, lin_b.reshape(1, C), acc[:, :N])


# speedup vs baseline: 25.5179x; 1.1701x over previous
import functools

import jax
import jax.numpy as jnp
from jax import lax
from jax.experimental import pallas as pl
from jax.experimental.pallas import tpu as pltpu
from jax.experimental.pallas import tpu_sc as plsc

N = 10000
E = 160000
C = 16
R = 64
NC = 2
NS = 16
L = 16
N2 = 10240
CH = 128
NCH = 79
EP = NCH * CH
E2 = NS * EP
NPT = N2 // NS
TRASH = N2 - 1

_MESH = plsc.VectorSubcoreMesh(
    core_axis_name="c", subcore_axis_name="s", num_cores=NC, num_subcores=NS
)
_SC_PARAMS = pltpu.CompilerParams(
    needs_layout_passes=False, use_tc_tiling_on_sc=False
)


@functools.partial(
    pl.kernel,
    out_type=jax.ShapeDtypeStruct((NC, NS, N2 // 128, 128), jnp.float32),
    mesh=_MESH,
    compiler_params=_SC_PARAMS,
    scratch_types=[
        pltpu.VMEM((NCH, CH), jnp.int32),
        pltpu.VMEM((N2 // 128, 128), jnp.float32),
    ],
)
def _sc_counts(ei_hbm, out_hbm, ev, cnt):
    c = lax.axis_index("c")
    s = lax.axis_index("s")
    pltpu.sync_copy(ei_hbm.at[c, s], ev)

    zeros = jnp.zeros((L,), jnp.float32)

    def _zero(i, carry):
        r = i // (128 // L)
        p = i % (128 // L)
        cnt[r, pl.ds(p * L, L)] = zeros
        return carry

    lax.fori_loop(0, N2 // L, _zero, 0)

    ones = jnp.ones((L,), jnp.float32)

    def _accum(i, carry):
        r = i // (CH // L)
        p = i % (CH // L)
        idx = ev[r, pl.ds(p * L, L)]
        plsc.addupdate_scatter(cnt, [idx >> 7, idx & 127], ones)
        return carry

    lax.fori_loop(0, EP // L, _accum, 0)
    pltpu.sync_copy(cnt, out_hbm.at[c, s])


def _norm_body(wf_ref, wb_ref, of_ref, ob_ref):
    for w_ref, o_ref in ((wf_ref, of_ref), (wb_ref, ob_ref)):
        w = w_ref[...]
        nrm = jnp.sqrt(jnp.sum(w * w, axis=1, keepdims=True)) + 0.01
        o_ref[...] = w / nrm


def _tc_norm(fwd_table, bwd_table):
    return pl.pallas_call(
        _norm_body,
        out_shape=(
            jax.ShapeDtypeStruct((R, C * C), jnp.float32),
            jax.ShapeDtypeStruct((R, C * C), jnp.float32),
        ),
    )(fwd_table, bwd_table)


BN = 1024


def _mm_body(x_ref, cp_ref, w2_ref, y_ref):
    x = x_ref[...]
    deg = jnp.sum(cp_ref[...], axis=1) + 1.0
    w2 = w2_ref[...]
    xf = x / deg[0][:, None]
    xb = x / deg[1][:, None]
    y_ref[0] = jnp.dot(xf, w2[0], preferred_element_type=jnp.float32)
    y_ref[1] = jnp.dot(xb, w2[1], preferred_element_type=jnp.float32)


def _tc_matmul(x, cntp, w2):
    return pl.pallas_call(
        _mm_body,
        grid=(N2 // BN,),
        in_specs=[
            pl.BlockSpec((BN, C), lambda i: (i, 0)),
            pl.BlockSpec((NC, NS, BN), lambda i: (0, 0, i)),
            pl.BlockSpec((NC, C, R * C), lambda i: (0, 0, 0)),
        ],
        out_specs=pl.BlockSpec((NC, BN, R * C), lambda i: (0, i, 0)),
        out_shape=jax.ShapeDtypeStruct((NC, N2, R * C), jnp.float32),
    )(x, cntp, w2)


@functools.partial(
    pl.kernel,
    out_type=jax.ShapeDtypeStruct((NC, N2, C), jnp.float32),
    mesh=_MESH,
    compiler_params=_SC_PARAMS,
    scratch_types=[
        pltpu.VMEM((NCH, CH), jnp.int32),
        pltpu.VMEM((NCH, CH), jnp.int32),
        pltpu.VMEM((NCH, CH), jnp.int32),
        pltpu.VMEM((CH, C), jnp.float32),
        pltpu.VMEM((CH, C), jnp.float32),
        pltpu.VMEM((CH, C), jnp.float32),
        pltpu.SemaphoreType.DMA,
        pltpu.SemaphoreType.DMA,
        pltpu.VMEM_SHARED((N2, C), jnp.float32),
    ],
)
def _sc_gather_scatter(ei_hbm, et_hbm, y_hbm, out_hbm,
                       gidx, didx, tt, rows0, rows1, zb, sem0, sem1, acc):
    c = lax.axis_index("c")
    s = lax.axis_index("s")

    zeros = jnp.zeros((L,), jnp.float32)

    def _zrow(r, carry):
        zb[r, :] = zeros
        return carry

    lax.fori_loop(0, CH, _zrow, 0)
    for k in range(NPT // CH):
        pltpu.sync_copy(zb, acc.at[pl.ds(s * NPT + k * CH, CH)])
    plsc.subcore_barrier()

    pltpu.sync_copy(ei_hbm.at[c, s], gidx)
    pltpu.sync_copy(ei_hbm.at[1 - c, s], didx)
    pltpu.sync_copy(et_hbm.at[s], tt)

    base = c * (N2 * R)

    def _build(r, carry):
        for p in range(CH // L):
            sl = pl.ds(p * L, L)
            gidx[r, sl] = gidx[r, sl] * R + tt[r, sl] + base
        return carry

    lax.fori_loop(0, NCH, _build, 0)

    pltpu.async_copy(y_hbm.at[gidx.at[0]], rows0, sem0)

    def _pair(k, carry):
        j = 2 * k
        pltpu.async_copy(y_hbm.at[gidx.at[j + 1]], rows1, sem1)
        pltpu.make_async_copy(y_hbm.at[gidx.at[j]], rows0, sem0).wait()
        pltpu.sync_copy(rows0, acc.at[didx.at[j]], add=True)
        pltpu.async_copy(y_hbm.at[gidx.at[j + 2]], rows0, sem0)
        pltpu.make_async_copy(y_hbm.at[gidx.at[j + 1]], rows1, sem1).wait()
        pltpu.sync_copy(rows1, acc.at[didx.at[j + 1]], add=True)
        return carry

    lax.fori_loop(0, (NCH - 1) // 2, _pair, 0)
    pltpu.make_async_copy(y_hbm.at[gidx.at[NCH - 1]], rows0, sem0).wait()
    pltpu.sync_copy(rows0, acc.at[didx.at[NCH - 1]], add=True)

    plsc.subcore_barrier()
    pltpu.sync_copy(acc.at[pl.ds(s * NPT, NPT)],
                    out_hbm.at[c, pl.ds(s * NPT, NPT)])


def _fin_body(x_ref, w_ref, b_ref, acc_ref, o_ref):
    x = x_ref[...]
    lin = lax.dot_general(x, w_ref[...], (((1,), (1,)), ((), ())),
                          preferred_element_type=jnp.float32)
    o_ref[...] = lin + b_ref[...] + acc_ref[0] + acc_ref[1]


def _tc_final(x, lin_w, lin_b2, acc):
    return pl.pallas_call(
        _fin_body,
        out_shape=jax.ShapeDtypeStruct((N, C), jnp.float32),
    )(x, lin_w, lin_b2, acc)


def kernel(x, edge_index, edge_type, lin_w, lin_b, fwd_table, bwd_table):
    pad = jnp.full((2, E2 - E), TRASH, jnp.int32)
    ei3 = jnp.concatenate([edge_index, pad], axis=1).reshape(NC, NS, NCH, CH)
    et3 = jnp.concatenate(
        [edge_type, jnp.zeros((E2 - E,), jnp.int32)]).reshape(NS, NCH, CH)

    cntp = _sc_counts(ei3)

    wnf, wnb = _tc_norm(fwd_table, bwd_table)
    w2 = jnp.stack([
        wnf.reshape(R, C, C).transpose(1, 0, 2).reshape(C, R * C),
        wnb.reshape(R, C, C).transpose(1, 0, 2).reshape(C, R * C),
    ])

    x2 = jnp.concatenate([x, jnp.zeros((N2 - N, C), x.dtype)])
    y = _tc_matmul(x2, cntp.reshape(NC, NS, N2), w2)
    y2 = y.reshape(NC * N2 * R, C)

    acc = _sc_gather_scatter(ei3, et3, y2)

    return _tc_final(x, lin_w, lin_b.reshape(1, C), acc[:, :N])


# speedup vs baseline: 28.5638x; 1.1194x over previous
import functools

import jax
import jax.numpy as jnp
from jax import lax
from jax.experimental import pallas as pl
from jax.experimental.pallas import tpu as pltpu
from jax.experimental.pallas import tpu_sc as plsc

N = 10000
E = 160000
C = 16
R = 64
NC = 2
NS = 16
L = 16
N2 = 10240
CH = 128
NCH = 79
EP = NCH * CH
E2 = NS * EP
NPT = N2 // NS
TRASH = N2 - 1

_MESH = plsc.VectorSubcoreMesh(
    core_axis_name="c", subcore_axis_name="s", num_cores=NC, num_subcores=NS
)
_SC_PARAMS = pltpu.CompilerParams(
    needs_layout_passes=False, use_tc_tiling_on_sc=False
)


@functools.partial(
    pl.kernel,
    out_type=jax.ShapeDtypeStruct((NC, NS, N2 // 128, 128), jnp.float32),
    mesh=_MESH,
    compiler_params=_SC_PARAMS,
    scratch_types=[
        pltpu.VMEM((NCH, CH), jnp.int32),
        pltpu.VMEM((N2 // 128, 128), jnp.float32),
    ],
)
def _sc_counts(ei_hbm, out_hbm, ev, cnt):
    c = lax.axis_index("c")
    s = lax.axis_index("s")
    pltpu.sync_copy(ei_hbm.at[c, s], ev)

    zeros = jnp.zeros((L,), jnp.float32)

    def _zero(i, carry):
        r = i // (128 // L)
        p = i % (128 // L)
        cnt[r, pl.ds(p * L, L)] = zeros
        return carry

    lax.fori_loop(0, N2 // L, _zero, 0)

    ones = jnp.ones((L,), jnp.float32)

    def _accum(i, carry):
        r = i // (CH // L)
        p = i % (CH // L)
        idx = ev[r, pl.ds(p * L, L)]
        plsc.addupdate_scatter(cnt, [idx >> 7, idx & 127], ones)
        return carry

    lax.fori_loop(0, EP // L, _accum, 0)
    pltpu.sync_copy(cnt, out_hbm.at[c, s])


def _norm_body(wf_ref, wb_ref, of_ref, ob_ref):
    for w_ref, o_ref in ((wf_ref, of_ref), (wb_ref, ob_ref)):
        w = w_ref[...]
        nrm = jnp.sqrt(jnp.sum(w * w, axis=1, keepdims=True)) + 0.01
        o_ref[...] = w / nrm


def _tc_norm(fwd_table, bwd_table):
    return pl.pallas_call(
        _norm_body,
        out_shape=(
            jax.ShapeDtypeStruct((R, C * C), jnp.float32),
            jax.ShapeDtypeStruct((R, C * C), jnp.float32),
        ),
    )(fwd_table, bwd_table)


BN = 1024


def _mm_body(x_ref, cp_ref, w2_ref, y_ref):
    x = x_ref[...]
    deg = jnp.sum(cp_ref[...], axis=1) + 1.0
    w2 = w2_ref[...]
    xf = x / deg[0][:, None]
    xb = x / deg[1][:, None]
    y_ref[0, 0] = jnp.dot(xf, w2[0], preferred_element_type=jnp.float32)
    y_ref[1, 0] = jnp.dot(xb, w2[1], preferred_element_type=jnp.float32)


def _tc_matmul(x, cntp, w2):
    return pl.pallas_call(
        _mm_body,
        grid=(N2 // BN, (R * C) // 128),
        in_specs=[
            pl.BlockSpec((BN, C), lambda i, g: (i, 0)),
            pl.BlockSpec((NC, NS, BN), lambda i, g: (0, 0, i)),
            pl.BlockSpec((NC, C, 128), lambda i, g: (0, 0, g)),
        ],
        out_specs=pl.BlockSpec((NC, 1, BN, 128), lambda i, g: (0, g, i, 0)),
        out_shape=jax.ShapeDtypeStruct((NC, (R * C) // 128, N2, 128),
                                       jnp.float32),
    )(x, cntp, w2)


@functools.partial(
    pl.kernel,
    out_type=jax.ShapeDtypeStruct((NC, N2, C), jnp.float32),
    mesh=_MESH,
    compiler_params=_SC_PARAMS,
    scratch_types=[
        pltpu.VMEM((NCH, CH), jnp.int32),
        pltpu.VMEM((NCH, CH), jnp.int32),
        pltpu.VMEM((NCH, CH), jnp.int32),
        pltpu.VMEM((CH, C), jnp.float32),
        pltpu.VMEM((CH, C), jnp.float32),
        pltpu.VMEM((CH, C), jnp.float32),
        pltpu.SemaphoreType.DMA,
        pltpu.SemaphoreType.DMA,
        pltpu.VMEM_SHARED((N2, C), jnp.float32),
    ],
)
def _sc_gather_scatter(ei_hbm, et_hbm, y_hbm, out_hbm,
                       gidx, didx, tt, rows0, rows1, zb, sem0, sem1, acc):
    c = lax.axis_index("c")
    s = lax.axis_index("s")

    zeros = jnp.zeros((L,), jnp.float32)

    def _zrow(r, carry):
        zb[r, :] = zeros
        return carry

    lax.fori_loop(0, CH, _zrow, 0)
    for k in range(NPT // CH):
        pltpu.sync_copy(zb, acc.at[pl.ds(s * NPT + k * CH, CH)])
    plsc.subcore_barrier()

    pltpu.sync_copy(ei_hbm.at[c, s], gidx)
    pltpu.sync_copy(ei_hbm.at[1 - c, s], didx)
    pltpu.sync_copy(et_hbm.at[s], tt)

    base = c * (N2 * R)

    def _build(r, carry):
        for p in range(CH // L):
            sl = pl.ds(p * L, L)
            tv = tt[r, sl]
            gidx[r, sl] = (gidx[r, sl] * 8 + (tv >> 3) * (N2 * 8)
                           + (tv & 7) + base)
        return carry

    lax.fori_loop(0, NCH, _build, 0)

    pltpu.async_copy(y_hbm.at[gidx.at[0]], rows0, sem0)

    def _pair(k, carry):
        j = 2 * k
        pltpu.async_copy(y_hbm.at[gidx.at[j + 1]], rows1, sem1)
        pltpu.make_async_copy(y_hbm.at[gidx.at[j]], rows0, sem0).wait()
        pltpu.sync_copy(rows0, acc.at[didx.at[j]], add=True)
        pltpu.async_copy(y_hbm.at[gidx.at[j + 2]], rows0, sem0)
        pltpu.make_async_copy(y_hbm.at[gidx.at[j + 1]], rows1, sem1).wait()
        pltpu.sync_copy(rows1, acc.at[didx.at[j + 1]], add=True)
        return carry

    lax.fori_loop(0, (NCH - 1) // 2, _pair, 0)
    pltpu.make_async_copy(y_hbm.at[gidx.at[NCH - 1]], rows0, sem0).wait()
    pltpu.sync_copy(rows0, acc.at[didx.at[NCH - 1]], add=True)

    plsc.subcore_barrier()
    pltpu.sync_copy(acc.at[pl.ds(s * NPT, NPT)],
                    out_hbm.at[c, pl.ds(s * NPT, NPT)])


def _fin_body(x_ref, w_ref, b_ref, acc_ref, o_ref):
    x = x_ref[...]
    lin = lax.dot_general(x, w_ref[...], (((1,), (1,)), ((), ())),
                          preferred_element_type=jnp.float32)
    o_ref[...] = lin + b_ref[...] + acc_ref[0] + acc_ref[1]


def _tc_final(x, lin_w, lin_b2, acc):
    return pl.pallas_call(
        _fin_body,
        out_shape=jax.ShapeDtypeStruct((N, C), jnp.float32),
    )(x, lin_w, lin_b2, acc)


def kernel(x, edge_index, edge_type, lin_w, lin_b, fwd_table, bwd_table):
    pad = jnp.full((2, E2 - E), TRASH, jnp.int32)
    ei3 = jnp.concatenate([edge_index, pad], axis=1).reshape(NC, NS, NCH, CH)
    et3 = jnp.concatenate(
        [edge_type, jnp.zeros((E2 - E,), jnp.int32)]).reshape(NS, NCH, CH)

    cntp = _sc_counts(ei3)

    wnf, wnb = _tc_norm(fwd_table, bwd_table)
    w2 = jnp.stack([
        wnf.reshape(R, C, C).transpose(1, 0, 2).reshape(C, R * C),
        wnb.reshape(R, C, C).transpose(1, 0, 2).reshape(C, R * C),
    ])

    x2 = jnp.concatenate([x, jnp.zeros((N2 - N, C), x.dtype)])
    y = _tc_matmul(x2, cntp.reshape(NC, NS, N2), w2)
    y2 = y.reshape(NC * N2 * R, C)

    acc = _sc_gather_scatter(ei3, et3, y2)

    return _tc_final(x, lin_w, lin_b.reshape(1, C), acc[:, :N])


# speedup vs baseline: 35.2027x; 1.2324x over previous
import functools

import jax
import jax.numpy as jnp
from jax import lax
from jax.experimental import pallas as pl
from jax.experimental.pallas import tpu as pltpu
from jax.experimental.pallas import tpu_sc as plsc

N = 10000
E = 160000
C = 16
R = 64
NC = 2
NS = 16
L = 16
N2 = 10240
CH = 128
NCH = 80
EP = NCH * CH
E2 = NS * EP
NPT = N2 // NS
TRASH = N2 - 1
NBUF = 4

_MESH = plsc.VectorSubcoreMesh(
    core_axis_name="c", subcore_axis_name="s", num_cores=NC, num_subcores=NS
)
_SC_PARAMS = pltpu.CompilerParams(
    needs_layout_passes=False, use_tc_tiling_on_sc=False
)


@functools.partial(
    pl.kernel,
    out_type=jax.ShapeDtypeStruct((NC, NS, N2 // 128, 128), jnp.float32),
    mesh=_MESH,
    compiler_params=_SC_PARAMS,
    scratch_types=[
        pltpu.VMEM((NCH, CH), jnp.int32),
        pltpu.VMEM((N2 // 128, 128), jnp.float32),
    ],
)
def _sc_counts(ei_hbm, out_hbm, ev, cnt):
    c = lax.axis_index("c")
    s = lax.axis_index("s")
    pltpu.sync_copy(ei_hbm.at[c, s], ev)

    zeros = jnp.zeros((L,), jnp.float32)

    def _zero(i, carry):
        r = i // (128 // L)
        p = i % (128 // L)
        cnt[r, pl.ds(p * L, L)] = zeros
        return carry

    lax.fori_loop(0, N2 // L, _zero, 0)

    ones = jnp.ones((L,), jnp.float32)

    def _accum(i, carry):
        r = i // (CH // L)
        p = i % (CH // L)
        idx = ev[r, pl.ds(p * L, L)]
        plsc.addupdate_scatter(cnt, [idx >> 7, idx & 127], ones)
        return carry

    lax.fori_loop(0, EP // L, _accum, 0)
    pltpu.sync_copy(cnt, out_hbm.at[c, s])


def _norm_body(wf_ref, wb_ref, of_ref, ob_ref):
    for w_ref, o_ref in ((wf_ref, of_ref), (wb_ref, ob_ref)):
        w = w_ref[...]
        nrm = jnp.sqrt(jnp.sum(w * w, axis=1, keepdims=True)) + 0.01
        o_ref[...] = w / nrm


def _tc_norm(fwd_table, bwd_table):
    return pl.pallas_call(
        _norm_body,
        out_shape=(
            jax.ShapeDtypeStruct((R, C * C), jnp.float32),
            jax.ShapeDtypeStruct((R, C * C), jnp.float32),
        ),
    )(fwd_table, bwd_table)


def _mm_body(x_ref, cp_ref, wbd_ref, y_ref, xs_ref):
    g = pl.program_id(0)

    @pl.when(g == 0)
    def _prep():
        x = x_ref[...]
        deg = jnp.sum(cp_ref[...], axis=1) + 1.0
        xs_ref[:, :C] = x / deg[0][:, None]
        xs_ref[:, C:] = x / deg[1][:, None]

    yy = jnp.dot(xs_ref[...], wbd_ref[0],
                 preferred_element_type=jnp.float32)
    y_ref[0, 0] = yy[:, :128]
    y_ref[1, 0] = yy[:, 128:]


def _tc_matmul(x, cntp, w2bd):
    return pl.pallas_call(
        _mm_body,
        grid=((R * C) // 128,),
        in_specs=[
            pl.BlockSpec((N2, C), lambda g: (0, 0)),
            pl.BlockSpec((NC, NS, N2), lambda g: (0, 0, 0)),
            pl.BlockSpec((1, 2 * C, 2 * 128), lambda g: (g, 0, 0)),
        ],
        out_specs=pl.BlockSpec((NC, 1, N2, 128), lambda g: (0, g, 0, 0)),
        out_shape=jax.ShapeDtypeStruct((NC, (R * C) // 128, N2, 128),
                                       jnp.float32),
        scratch_shapes=[pltpu.VMEM((N2, 2 * C), jnp.float32)],
    )(x, cntp, w2bd)


@functools.partial(
    pl.kernel,
    out_type=jax.ShapeDtypeStruct((NC, N2, C), jnp.float32),
    mesh=_MESH,
    compiler_params=_SC_PARAMS,
    scratch_types=[
        pltpu.VMEM((NCH, CH), jnp.int32),
        pltpu.VMEM((NCH, CH), jnp.int32),
        pltpu.VMEM((NCH, CH), jnp.int32),
        pltpu.VMEM((CH, C), jnp.float32),
        pltpu.VMEM((CH, C), jnp.float32),
        pltpu.VMEM((CH, C), jnp.float32),
        pltpu.SemaphoreType.DMA,
        pltpu.SemaphoreType.DMA,
        pltpu.VMEM_SHARED((N2, C), jnp.float32),
    ],
)
def _sc_gather_scatter(ei_hbm, et_hbm, y_hbm, out_hbm,
                       gidx, didx, tt, rows0, rows1, zb, sem0, sem1, acc):
    c = lax.axis_index("c")
    s = lax.axis_index("s")

    zeros = jnp.zeros((L,), jnp.float32)

    def _zrow(r, carry):
        zb[r, :] = zeros
        return carry

    lax.fori_loop(0, CH, _zrow, 0)
    for k in range(NPT // CH):
        pltpu.sync_copy(zb, acc.at[pl.ds(s * NPT + k * CH, CH)])
    plsc.subcore_barrier()

    pltpu.sync_copy(ei_hbm.at[c, s], gidx)
    pltpu.sync_copy(ei_hbm.at[1 - c, s], didx)
    pltpu.sync_copy(et_hbm.at[s], tt)

    base = c * (N2 * R)

    def _build(r, carry):
        for p in range(CH // L):
            sl = pl.ds(p * L, L)
            tv = tt[r, sl]
            gidx[r, sl] = (gidx[r, sl] * 8 + (tv >> 3) * (N2 * 8)
                           + (tv & 7) + base)
        return carry

    lax.fori_loop(0, NCH, _build, 0)

    pltpu.async_copy(y_hbm.at[gidx.at[0]], rows0, sem0)

    def _pair(k, carry):
        j = 2 * k
        pltpu.async_copy(y_hbm.at[gidx.at[j + 1]], rows1, sem1)
        pltpu.make_async_copy(y_hbm.at[gidx.at[j]], rows0, sem0).wait()
        pltpu.sync_copy(rows0, acc.at[didx.at[j]], add=True)
        pltpu.async_copy(y_hbm.at[gidx.at[j + 2]], rows0, sem0)
        pltpu.make_async_copy(y_hbm.at[gidx.at[j + 1]], rows1, sem1).wait()
        pltpu.sync_copy(rows1, acc.at[didx.at[j + 1]], add=True)
        return carry

    lax.fori_loop(0, NCH // 2 - 1, _pair, 0)
    pltpu.make_async_copy(y_hbm.at[gidx.at[NCH - 2]], rows0, sem0).wait()
    pltpu.sync_copy(rows0, acc.at[didx.at[NCH - 2]], add=True)
    pltpu.async_copy(y_hbm.at[gidx.at[NCH - 1]], rows1, sem1)
    pltpu.make_async_copy(y_hbm.at[gidx.at[NCH - 1]], rows1, sem1).wait()
    pltpu.sync_copy(rows1, acc.at[didx.at[NCH - 1]], add=True)

    plsc.subcore_barrier()
    pltpu.sync_copy(acc.at[pl.ds(s * NPT, NPT)],
                    out_hbm.at[c, pl.ds(s * NPT, NPT)])


def _fin_body(x_ref, w_ref, b_ref, acc_ref, o_ref):
    x = x_ref[...]
    lin = lax.dot_general(x, w_ref[...], (((1,), (1,)), ((), ())),
                          preferred_element_type=jnp.float32)
    o_ref[...] = lin + b_ref[...] + acc_ref[0, :N, :] + acc_ref[1, :N, :]


def _tc_final(x, lin_w, lin_b2, acc):
    return pl.pallas_call(
        _fin_body,
        out_shape=jax.ShapeDtypeStruct((N, C), jnp.float32),
    )(x, lin_w, lin_b2, acc)


def kernel(x, edge_index, edge_type, lin_w, lin_b, fwd_table, bwd_table):
    pad = jnp.full((2, E2 - E), TRASH, jnp.int32)
    ei3 = jnp.concatenate([edge_index, pad], axis=1).reshape(NC, NS, NCH, CH)
    et3 = jnp.concatenate(
        [edge_type, jnp.zeros((E2 - E,), jnp.int32)]).reshape(NS, NCH, CH)

    cntp = _sc_counts(ei3).reshape(NC, NS, N2)

    wnf, wnb = _tc_norm(fwd_table, bwd_table)
    w2f = wnf.reshape(R, C, C).transpose(1, 0, 2).reshape(C, 8, 128)
    w2b = wnb.reshape(R, C, C).transpose(1, 0, 2).reshape(C, 8, 128)
    w2bd = (
        jnp.zeros((8, 2 * C, 2 * 128), jnp.float32)
        .at[:, :C, :128].set(w2f.transpose(1, 0, 2))
        .at[:, C:, 128:].set(w2b.transpose(1, 0, 2))
    )

    x2 = jnp.concatenate([x, jnp.zeros((N2 - N, C), x.dtype)])
    y = _tc_matmul(x2, cntp, w2bd)
    y2 = y.reshape(NC * N2 * R, C)

    acc = _sc_gather_scatter(ei3, et3, y2)

    return _tc_final(x, lin_w, lin_b.reshape(1, C), acc)


# speedup vs baseline: 37.7378x; 1.0720x over previous
import functools

import jax
import jax.numpy as jnp
from jax import lax
from jax.experimental import pallas as pl
from jax.experimental.pallas import tpu as pltpu
from jax.experimental.pallas import tpu_sc as plsc

N = 10000
E = 160000
C = 16
R = 64
NC = 2
NS = 16
L = 16
N2 = 10240
CH = 128
NCH = 80
EP = NCH * CH
E2 = NS * EP
NPT = N2 // NS
TRASH = N2 - 1
NBUF = 4

_MESH = plsc.VectorSubcoreMesh(
    core_axis_name="c", subcore_axis_name="s", num_cores=NC, num_subcores=NS
)
_SC_PARAMS = pltpu.CompilerParams(
    needs_layout_passes=False, use_tc_tiling_on_sc=False
)


@functools.partial(
    pl.kernel,
    out_type=jax.ShapeDtypeStruct((NC, NS, N2 // 128, 128), jnp.float32),
    mesh=_MESH,
    compiler_params=_SC_PARAMS,
    scratch_types=[
        pltpu.VMEM((NCH, CH), jnp.int32),
        pltpu.VMEM((N2 // 128, 128), jnp.float32),
    ],
)
def _sc_counts(ei_hbm, out_hbm, ev, cnt):
    c = lax.axis_index("c")
    s = lax.axis_index("s")
    pltpu.sync_copy(ei_hbm.at[c, s], ev)

    zeros = jnp.zeros((L,), jnp.float32)

    def _zero(i, carry):
        r = i // (128 // L)
        p = i % (128 // L)
        cnt[r, pl.ds(p * L, L)] = zeros
        return carry

    lax.fori_loop(0, N2 // L, _zero, 0)

    ones = jnp.ones((L,), jnp.float32)

    def _accum(i, carry):
        r = i // (CH // L)
        p = i % (CH // L)
        idx = ev[r, pl.ds(p * L, L)]
        plsc.addupdate_scatter(cnt, [idx >> 7, idx & 127], ones)
        return carry

    lax.fori_loop(0, EP // L, _accum, 0)
    pltpu.sync_copy(cnt, out_hbm.at[c, s])


def _norm_body(wf_ref, wb_ref, of_ref, ob_ref):
    for w_ref, o_ref in ((wf_ref, of_ref), (wb_ref, ob_ref)):
        w = w_ref[...]
        nrm = jnp.sqrt(jnp.sum(w * w, axis=1, keepdims=True)) + 0.01
        o_ref[...] = w / nrm


def _tc_norm(fwd_table, bwd_table):
    return pl.pallas_call(
        _norm_body,
        out_shape=(
            jax.ShapeDtypeStruct((R, C * C), jnp.float32),
            jax.ShapeDtypeStruct((R, C * C), jnp.float32),
        ),
    )(fwd_table, bwd_table)


def _mm_body(x_ref, cp_ref, wbd_ref, y_ref, xs_ref):
    g = pl.program_id(0)

    @pl.when(g == 0)
    def _prep():
        x = x_ref[...]
        deg = jnp.sum(cp_ref[...], axis=1) + 1.0
        xs_ref[:, :C] = x / deg[0][:, None]
        xs_ref[:, C:] = x / deg[1][:, None]

    yy = jnp.dot(xs_ref[...], wbd_ref[0],
                 preferred_element_type=jnp.float32)
    y_ref[0, 0] = yy[:, :128]
    y_ref[1, 0] = yy[:, 128:]


def _tc_matmul(x, cntp, w2bd):
    return pl.pallas_call(
        _mm_body,
        grid=((R * C) // 128,),
        in_specs=[
            pl.BlockSpec((N2, C), lambda g: (0, 0)),
            pl.BlockSpec((NC, NS, N2), lambda g: (0, 0, 0)),
            pl.BlockSpec((1, 2 * C, 2 * 128), lambda g: (g, 0, 0)),
        ],
        out_specs=pl.BlockSpec((NC, 1, N2, 128), lambda g: (0, g, 0, 0)),
        out_shape=jax.ShapeDtypeStruct((NC, (R * C) // 128, N2, 128),
                                       jnp.float32),
        scratch_shapes=[pltpu.VMEM((N2, 2 * C), jnp.float32)],
    )(x, cntp, w2bd)


@functools.partial(
    pl.kernel,
    out_type=jax.ShapeDtypeStruct((NC, N2, C), jnp.float32),
    mesh=_MESH,
    compiler_params=_SC_PARAMS,
    scratch_types=[
        pltpu.VMEM((NCH, CH), jnp.int32),
        pltpu.VMEM((NCH, CH), jnp.int32),
        pltpu.VMEM((NCH, CH), jnp.int32),
        pltpu.VMEM((CH, C), jnp.float32),
        pltpu.VMEM((CH, C), jnp.float32),
        pltpu.VMEM((CH, C), jnp.float32),
        pltpu.VMEM((CH, C), jnp.float32),
        pltpu.VMEM((CH, C), jnp.float32),
        pltpu.SemaphoreType.DMA,
        pltpu.SemaphoreType.DMA,
        pltpu.SemaphoreType.DMA,
        pltpu.SemaphoreType.DMA,
        pltpu.VMEM_SHARED((N2, C), jnp.float32),
    ],
)
def _sc_gather_scatter(ei_hbm, et_hbm, y_hbm, out_hbm,
                       gidx, didx, tt, rows0, rows1, rows2, rows3, zb,
                       sem0, sem1, sem2, sem3, acc):
    c = lax.axis_index("c")
    s = lax.axis_index("s")

    zeros = jnp.zeros((L,), jnp.float32)

    def _zrow(r, carry):
        zb[r, :] = zeros
        return carry

    lax.fori_loop(0, CH, _zrow, 0)
    for k in range(NPT // CH):
        pltpu.sync_copy(zb, acc.at[pl.ds(s * NPT + k * CH, CH)])
    plsc.subcore_barrier()

    pltpu.sync_copy(ei_hbm.at[c, s], gidx)
    pltpu.sync_copy(ei_hbm.at[1 - c, s], didx)
    pltpu.sync_copy(et_hbm.at[s], tt)

    base = c * (N2 * R)

    def _build(r, carry):
        for p in range(CH // L):
            sl = pl.ds(p * L, L)
            tv = tt[r, sl]
            gidx[r, sl] = (gidx[r, sl] * 8 + (tv >> 3) * (N2 * 8)
                           + (tv & 7) + base)
        return carry

    lax.fori_loop(0, NCH, _build, 0)

    bufs = ((rows0, sem0), (rows1, sem1), (rows2, sem2), (rows3, sem3))
    for b, (rw, sm) in enumerate(bufs):
        pltpu.async_copy(y_hbm.at[gidx.at[b]], rw, sm)

    def _quad(k, carry):
        j0 = NBUF * k
        for b, (rw, sm) in enumerate(bufs):
            j = j0 + b
            pltpu.make_async_copy(y_hbm.at[gidx.at[j]], rw, sm).wait()
            pltpu.sync_copy(rw, acc.at[didx.at[j]], add=True)
            pltpu.async_copy(y_hbm.at[gidx.at[j + NBUF]], rw, sm)
        return carry

    lax.fori_loop(0, NCH // NBUF - 1, _quad, 0)
    for b, (rw, sm) in enumerate(bufs):
        j = NCH - NBUF + b
        pltpu.make_async_copy(y_hbm.at[gidx.at[j]], rw, sm).wait()
        pltpu.sync_copy(rw, acc.at[didx.at[j]], add=True)

    plsc.subcore_barrier()
    pltpu.sync_copy(acc.at[pl.ds(s * NPT, NPT)],
                    out_hbm.at[c, pl.ds(s * NPT, NPT)])


def _fin_body(x_ref, w_ref, b_ref, acc_ref, o_ref):
    x = x_ref[...]
    lin = lax.dot_general(x, w_ref[...], (((1,), (1,)), ((), ())),
                          preferred_element_type=jnp.float32)
    o_ref[...] = lin + b_ref[...] + acc_ref[0, :N, :] + acc_ref[1, :N, :]


def _tc_final(x, lin_w, lin_b2, acc):
    return pl.pallas_call(
        _fin_body,
        out_shape=jax.ShapeDtypeStruct((N, C), jnp.float32),
    )(x, lin_w, lin_b2, acc)


def kernel(x, edge_index, edge_type, lin_w, lin_b, fwd_table, bwd_table):
    pad = jnp.full((2, E2 - E), TRASH, jnp.int32)
    ei3 = jnp.concatenate([edge_index, pad], axis=1).reshape(NC, NS, NCH, CH)
    et3 = jnp.concatenate(
        [edge_type, jnp.zeros((E2 - E,), jnp.int32)]).reshape(NS, NCH, CH)

    cntp = _sc_counts(ei3).reshape(NC, NS, N2)

    wnf, wnb = _tc_norm(fwd_table, bwd_table)
    w2f = wnf.reshape(R, C, C).transpose(1, 0, 2).reshape(C, 8, 128)
    w2b = wnb.reshape(R, C, C).transpose(1, 0, 2).reshape(C, 8, 128)
    w2bd = (
        jnp.zeros((8, 2 * C, 2 * 128), jnp.float32)
        .at[:, :C, :128].set(w2f.transpose(1, 0, 2))
        .at[:, C:, 128:].set(w2b.transpose(1, 0, 2))
    )

    x2 = jnp.concatenate([x, jnp.zeros((N2 - N, C), x.dtype)])
    y = _tc_matmul(x2, cntp, w2bd)
    y2 = y.reshape(NC * N2 * R, C)

    acc = _sc_gather_scatter(ei3, et3, y2)

    return _tc_final(x, lin_w, lin_b.reshape(1, C), acc)
